# Initial kernel scaffold; baseline (speedup 1.0000x reference)
import jax
import jax.numpy as jnp
from jax.experimental import pallas as pl


def kernel(x, edge_index, edge_attr, batch, W_in, b_in, W1, b1, W2, b2, Wu, bu, W_pred, b_pred):
    raise NotImplementedError("write your pallas kernel here")


# speedup vs baseline: 2.1246x; 2.1246x over previous
import functools

import jax
import jax.numpy as jnp
from jax import lax
from jax.experimental import pallas as pl
from jax.experimental.pallas import tpu as pltpu
from jax.experimental.pallas import tpu_sc as plsc

N = 10000
E = 320000
D = 128
L = 4
G = 64

CHUNK = 128
NCHUNK = E // CHUNK
NWORK = 32
CPW = NCHUNK // NWORK
CPS = (NCHUNK // 2) // 16
ROWS_BIG = 632
ROWS_LAST = N - 15 * ROWS_BIG


def _lrelu(v):
    return jnp.where(v >= 0, v, 0.01 * v)


def _node_init_body(x_ref, win_ref, bin_ref, wa_ref, wb_ref, h_ref, a_ref, b_ref):
    h = jnp.dot(x_ref[...], win_ref[...], preferred_element_type=jnp.float32)
    h = h + bin_ref[...]
    h_ref[...] = h
    a_ref[...] = jnp.dot(h, wa_ref[...], preferred_element_type=jnp.float32)
    b_ref[...] = jnp.dot(h, wb_ref[...], preferred_element_type=jnp.float32)


def _node_init(x, w_in, b_in, wa, wb):
    nb = 2000
    grid = N // nb
    blk_n = pl.BlockSpec((nb, D), lambda i: (i, 0))
    blk_w = pl.BlockSpec((D, D), lambda i: (0, 0))
    blk_b = pl.BlockSpec((1, D), lambda i: (0, 0))
    out = jax.ShapeDtypeStruct((N, D), jnp.float32)
    return pl.pallas_call(
        _node_init_body,
        grid=(grid,),
        in_specs=[blk_n, blk_w, blk_b, blk_w, blk_w],
        out_specs=[blk_n, blk_n, blk_n],
        out_shape=[out, out, out],
    )(x, w_in, b_in, wa, wb)


def _edge_body(gs_ref, ea_ref, w1c_ref, b1_ref, w2_ref, b2_ref, u_ref):
    t = gs_ref[...] + jnp.dot(ea_ref[...], w1c_ref[...],
                              preferred_element_type=jnp.float32)
    t = _lrelu(t + b1_ref[...])
    u = jnp.dot(t, w2_ref[...], preferred_element_type=jnp.float32)
    u_ref[...] = _lrelu(u + b2_ref[...])


def _edge_mlp(gsum, ea, w1c, b1l, w2l, b2l):
    be = 512
    grid = E // be
    blk_e = pl.BlockSpec((be, D), lambda i: (i, 0))
    blk_w = pl.BlockSpec((D, D), lambda i: (0, 0))
    blk_b = pl.BlockSpec((1, D), lambda i: (0, 0))
    return pl.pallas_call(
        _edge_body,
        grid=(grid,),
        in_specs=[blk_e, blk_e, blk_w, blk_b, blk_w, blk_b],
        out_specs=blk_e,
        out_shape=jax.ShapeDtypeStruct((E, D), jnp.float32),
    )(gsum, ea, w1c, b1l, w2l, b2l)


def _node_upd_body(h_ref, p_ref, wua_ref, wub_ref, bu_ref, wa_ref, wb_ref,
                   hn_ref, a_ref, b_ref):
    aggr = p_ref[0] + p_ref[1]
    upd = jnp.dot(h_ref[...], wua_ref[...], preferred_element_type=jnp.float32)
    upd = upd + jnp.dot(aggr, wub_ref[...], preferred_element_type=jnp.float32)
    hn = h_ref[...] + _lrelu(upd + bu_ref[...])
    hn_ref[...] = hn
    if a_ref is not None:
        a_ref[...] = jnp.dot(hn, wa_ref[...], preferred_element_type=jnp.float32)
        b_ref[...] = jnp.dot(hn, wb_ref[...], preferred_element_type=jnp.float32)


def _node_update(h, parts, wua, wub, bul, wa_next, wb_next):
    nb = 2000
    grid = N // nb
    blk_n = pl.BlockSpec((nb, D), lambda i: (i, 0))
    blk_p = pl.BlockSpec((2, nb, D), lambda i: (0, i, 0))
    blk_w = pl.BlockSpec((D, D), lambda i: (0, 0))
    blk_b = pl.BlockSpec((1, D), lambda i: (0, 0))
    out = jax.ShapeDtypeStruct((N, D), jnp.float32)
    if wa_next is None:
        body = lambda h_r, p_r, wua_r, wub_r, bu_r, hn_r: _node_upd_body(
            h_r, p_r, wua_r, wub_r, bu_r, None, None, hn_r, None, None)
        return pl.pallas_call(
            body,
            grid=(grid,),
            in_specs=[blk_n, blk_p, blk_w, blk_w, blk_b],
            out_specs=blk_n,
            out_shape=out,
        )(h, parts, wua, wub, bul)
    return pl.pallas_call(
        _node_upd_body,
        grid=(grid,),
        in_specs=[blk_n, blk_p, blk_w, blk_w, blk_b, blk_w, blk_w],
        out_specs=[blk_n, blk_n, blk_n],
        out_shape=[out, out, out],
    )(h, parts, wua, wub, bul, wa_next, wb_next)


def _pool_body(h_ref, batch_ref, wpt_ref, bp_ref, out_ref):
    seg = lax.broadcasted_iota(jnp.int32, (G, N), 0)
    oh = (seg == batch_ref[...]).astype(jnp.float32)
    sums = jnp.dot(oh, h_ref[...], preferred_element_type=jnp.float32)
    cnts = jnp.sum(oh, axis=1, keepdims=True)
    hg = sums / jnp.maximum(cnts, 1.0)
    out_ref[...] = jnp.sum(hg * wpt_ref[...], axis=1, keepdims=True) + bp_ref[...]


def _pool(h, batch2d, wpt, bp):
    return pl.pallas_call(
        _pool_body,
        out_shape=jax.ShapeDtypeStruct((G, 1), jnp.float32),
    )(h, batch2d, wpt, bp)


def _sc_mesh():
    return plsc.VectorSubcoreMesh(core_axis_name="c", subcore_axis_name="s",
                                  num_cores=2, num_subcores=16)


def _sc_gather_body(a_hbm, b_hbm, dst_hbm, src_hbm, out_hbm,
                    di_v, si_v, ar_v, br_v, sem_a, sem_b):
    c = lax.axis_index("c")
    s = lax.axis_index("s")
    w = c * 16 + s

    def do_chunk(chunk):
        pltpu.sync_copy(dst_hbm.at[chunk], di_v)
        pltpu.sync_copy(src_hbm.at[chunk], si_v)
        cp_a = pltpu.async_copy(a_hbm.at[di_v], ar_v, sem_a)
        cp_b = pltpu.async_copy(b_hbm.at[si_v], br_v, sem_b)
        cp_a.wait()
        cp_b.wait()

        def add_row(r, carry):
            for cc in range(8):
                sl = pl.ds(cc * 16, 16)
                ar_v[r, sl] = ar_v[r, sl] + br_v[r, sl]
            return carry

        lax.fori_loop(0, CHUNK, add_row, 0)
        pltpu.sync_copy(ar_v, out_hbm.at[chunk])

    def loop_body(i, carry):
        do_chunk(w * CPW + i)
        return carry

    lax.fori_loop(0, CPW, loop_body, 0)

    @pl.when(w < NCHUNK - NWORK * CPW)
    def _():
        do_chunk(NWORK * CPW + w)


def _sc_gather(a, b, dst_c, src_c):
    kern = pl.kernel(
        _sc_gather_body,
        out_type=jax.ShapeDtypeStruct((NCHUNK, CHUNK, D), jnp.float32),
        mesh=_sc_mesh(),
        scratch_types=[
            pltpu.VMEM((CHUNK,), jnp.int32),
            pltpu.VMEM((CHUNK,), jnp.int32),
            pltpu.VMEM((CHUNK, D), jnp.float32),
            pltpu.VMEM((CHUNK, D), jnp.float32),
            pltpu.SemaphoreType.DMA,
            pltpu.SemaphoreType.DMA,
        ],
    )
    return kern(a, b, dst_c, src_c)


def _sc_scatter_body(u_hbm, dst_hbm, out_hbm, di_v, ur_v, z_v, acc_sh):
    c = lax.axis_index("c")
    s = lax.axis_index("s")

    def zrow(r, carry):
        for cc in range(8):
            z_v[r, pl.ds(cc * 16, 16)] = jnp.zeros((16,), jnp.float32)
        return carry

    lax.fori_loop(0, 8, zrow, 0)
    base_rows = pl.multiple_of(s * ROWS_BIG, 8)
    nz = jnp.where(s < 15, ROWS_BIG // 8, ROWS_LAST // 8)

    def zcopy(i, carry):
        pltpu.sync_copy(z_v, acc_sh.at[pl.ds(base_rows + pl.multiple_of(i * 8, 8), 8)])
        return carry

    lax.fori_loop(0, nz, zcopy, 0)
    plsc.subcore_barrier()

    half = NCHUNK // 2

    def do_chunk(chunk):
        pltpu.sync_copy(dst_hbm.at[chunk], di_v)
        pltpu.sync_copy(u_hbm.at[chunk], ur_v)
        pltpu.sync_copy(ur_v, acc_sh.at[di_v], add=True)

    def loop_body(i, carry):
        do_chunk(c * half + s * CPS + i)
        return carry

    lax.fori_loop(0, CPS, loop_body, 0)

    @pl.when(s < half - 16 * CPS)
    def _():
        do_chunk(c * half + 16 * CPS + s)

    plsc.subcore_barrier()
    out_base = pl.multiple_of(c * N + s * ROWS_BIG, 8)

    @pl.when(s < 15)
    def _():
        pltpu.sync_copy(acc_sh.at[pl.ds(base_rows, ROWS_BIG)],
                        out_hbm.at[pl.ds(out_base, ROWS_BIG)])

    @pl.when(s == 15)
    def _():
        pltpu.sync_copy(acc_sh.at[pl.ds(base_rows, ROWS_LAST)],
                        out_hbm.at[pl.ds(out_base, ROWS_LAST)])


def _sc_scatter(u_c, dst_c):
    kern = pl.kernel(
        _sc_scatter_body,
        out_type=jax.ShapeDtypeStruct((2 * N, D), jnp.float32),
        mesh=_sc_mesh(),
        scratch_types=[
            pltpu.VMEM((CHUNK,), jnp.int32),
            pltpu.VMEM((CHUNK, D), jnp.float32),
            pltpu.VMEM((8, D), jnp.float32),
            pltpu.VMEM_SHARED((N, D), jnp.float32),
        ],
    )
    return kern(u_c, dst_c)


def kernel(x, edge_index, edge_attr, batch, W_in, b_in, W1, b1, W2, b2,
           Wu, bu, W_pred, b_pred):
    src_c = edge_index[0].reshape(NCHUNK, CHUNK)
    dst_c = edge_index[1].reshape(NCHUNK, CHUNK)
    batch2d = batch.reshape(1, N)

    h, a, b = _node_init(x, W_in, b_in.reshape(1, D),
                         W1[0, :D, :], W1[0, D:2 * D, :])
    for l in range(L):
        gsum = _sc_gather(a, b, dst_c, src_c)
        u = _edge_mlp(gsum.reshape(E, D), edge_attr, W1[l, 2 * D:, :],
                      b1[l].reshape(1, D), W2[l], b2[l].reshape(1, D))
        parts = _sc_scatter(u.reshape(NCHUNK, CHUNK, D), dst_c)
        parts = parts.reshape(2, N, D)
        if l + 1 < L:
            h, a, b = _node_update(h, parts, Wu[l, :D, :], Wu[l, D:, :],
                                   bu[l].reshape(1, D),
                                   W1[l + 1, :D, :], W1[l + 1, D:2 * D, :])
        else:
            h = _node_update(h, parts, Wu[l, :D, :], Wu[l, D:, :],
                             bu[l].reshape(1, D), None, None)

    out = _pool(h, batch2d, W_pred.reshape(1, D), b_pred.reshape(1, 1))
    return out.reshape(-1)


# speedup vs baseline: 2.4937x; 1.1737x over previous
import functools

import jax
import jax.numpy as jnp
from jax import lax
from jax.experimental import pallas as pl
from jax.experimental.pallas import tpu as pltpu
from jax.experimental.pallas import tpu_sc as plsc

N = 10000
E = 320000
D = 128
L = 4
G = 64

CHUNK = 128
NCHUNK = E // CHUNK
NWORK = 32
CPW = NCHUNK // NWORK
CPS = (NCHUNK // 2) // 16
ROWS_BIG = 632
ROWS_LAST = N - 15 * ROWS_BIG


def _lrelu(v):
    return jnp.where(v >= 0, v, 0.01 * v)


def _node_init_body(x_ref, win_ref, bin_ref, wa_ref, wb_ref, h_ref, a_ref, b_ref):
    h = jnp.dot(x_ref[...], win_ref[...], preferred_element_type=jnp.float32)
    h = h + bin_ref[...]
    h_ref[...] = h
    a_ref[...] = jnp.dot(h, wa_ref[...], preferred_element_type=jnp.float32)
    b_ref[...] = jnp.dot(h, wb_ref[...], preferred_element_type=jnp.float32)


def _node_init(x, w_in, b_in, wa, wb):
    nb = 2000
    grid = N // nb
    blk_n = pl.BlockSpec((nb, D), lambda i: (i, 0))
    blk_w = pl.BlockSpec((D, D), lambda i: (0, 0))
    blk_b = pl.BlockSpec((1, D), lambda i: (0, 0))
    out = jax.ShapeDtypeStruct((N, D), jnp.float32)
    return pl.pallas_call(
        _node_init_body,
        grid=(grid,),
        in_specs=[blk_n, blk_w, blk_b, blk_w, blk_w],
        out_specs=[blk_n, blk_n, blk_n],
        out_shape=[out, out, out],
    )(x, w_in, b_in, wa, wb)


def _edge_body(gs_ref, ea_ref, w1c_ref, b1_ref, w2_ref, b2_ref, u_ref):
    t = gs_ref[...] + jnp.dot(ea_ref[...], w1c_ref[...],
                              preferred_element_type=jnp.float32)
    t = _lrelu(t + b1_ref[...])
    u = jnp.dot(t, w2_ref[...], preferred_element_type=jnp.float32)
    u_ref[...] = _lrelu(u + b2_ref[...])


def _edge_mlp(gsum, ea, w1c, b1l, w2l, b2l):
    be = 512
    grid = E // be
    blk_e = pl.BlockSpec((be, D), lambda i: (i, 0))
    blk_w = pl.BlockSpec((D, D), lambda i: (0, 0))
    blk_b = pl.BlockSpec((1, D), lambda i: (0, 0))
    return pl.pallas_call(
        _edge_body,
        grid=(grid,),
        in_specs=[blk_e, blk_e, blk_w, blk_b, blk_w, blk_b],
        out_specs=blk_e,
        out_shape=jax.ShapeDtypeStruct((E, D), jnp.float32),
    )(gsum, ea, w1c, b1l, w2l, b2l)


def _node_upd_body(h_ref, p_ref, wua_ref, wub_ref, bu_ref, wa_ref, wb_ref,
                   hn_ref, a_ref, b_ref):
    aggr = p_ref[0] + p_ref[1]
    upd = jnp.dot(h_ref[...], wua_ref[...], preferred_element_type=jnp.float32)
    upd = upd + jnp.dot(aggr, wub_ref[...], preferred_element_type=jnp.float32)
    hn = h_ref[...] + _lrelu(upd + bu_ref[...])
    hn_ref[...] = hn
    if a_ref is not None:
        a_ref[...] = jnp.dot(hn, wa_ref[...], preferred_element_type=jnp.float32)
        b_ref[...] = jnp.dot(hn, wb_ref[...], preferred_element_type=jnp.float32)


def _node_update(h, parts, wua, wub, bul, wa_next, wb_next):
    nb = 2000
    grid = N // nb
    blk_n = pl.BlockSpec((nb, D), lambda i: (i, 0))
    blk_p = pl.BlockSpec((2, nb, D), lambda i: (0, i, 0))
    blk_w = pl.BlockSpec((D, D), lambda i: (0, 0))
    blk_b = pl.BlockSpec((1, D), lambda i: (0, 0))
    out = jax.ShapeDtypeStruct((N, D), jnp.float32)
    if wa_next is None:
        body = lambda h_r, p_r, wua_r, wub_r, bu_r, hn_r: _node_upd_body(
            h_r, p_r, wua_r, wub_r, bu_r, None, None, hn_r, None, None)
        return pl.pallas_call(
            body,
            grid=(grid,),
            in_specs=[blk_n, blk_p, blk_w, blk_w, blk_b],
            out_specs=blk_n,
            out_shape=out,
        )(h, parts, wua, wub, bul)
    return pl.pallas_call(
        _node_upd_body,
        grid=(grid,),
        in_specs=[blk_n, blk_p, blk_w, blk_w, blk_b, blk_w, blk_w],
        out_specs=[blk_n, blk_n, blk_n],
        out_shape=[out, out, out],
    )(h, parts, wua, wub, bul, wa_next, wb_next)


def _pool_body(h_ref, batch_ref, wpt_ref, bp_ref, out_ref):
    seg = lax.broadcasted_iota(jnp.int32, (G, N), 0)
    oh = (seg == batch_ref[...]).astype(jnp.float32)
    h32 = h_ref[...]
    h_hi = h32.astype(jnp.bfloat16).astype(jnp.float32)
    h_lo = h32 - h_hi
    sums = (jnp.dot(oh, h_hi, preferred_element_type=jnp.float32)
            + jnp.dot(oh, h_lo, preferred_element_type=jnp.float32))
    cnts = jnp.sum(oh, axis=1, keepdims=True)
    hg = sums / jnp.maximum(cnts, 1.0)
    hgb = hg.astype(jnp.bfloat16).astype(jnp.float32)
    wptb = wpt_ref[...].astype(jnp.bfloat16).astype(jnp.float32)
    out_ref[...] = jnp.sum(hgb * wptb, axis=1, keepdims=True) + bp_ref[...]


def _pool(h, batch2d, wpt, bp):
    return pl.pallas_call(
        _pool_body,
        out_shape=jax.ShapeDtypeStruct((G, 1), jnp.float32),
    )(h, batch2d, wpt, bp)


def _sc_mesh():
    return plsc.VectorSubcoreMesh(core_axis_name="c", subcore_axis_name="s",
                                  num_cores=2, num_subcores=16)


def _sc_gather_body(a_hbm, b_hbm, dst_hbm, src_hbm, out_hbm,
                    di0, si0, ar0, br0, di1, si1, ar1, br1,
                    sa0, sb0, sa1, sb1):
    c = lax.axis_index("c")
    s = lax.axis_index("s")
    w = c * 16 + s

    bufs = ((di0, si0, ar0, br0, sa0, sb0), (di1, si1, ar1, br1, sa1, sb1))

    def start(chunk, j):
        di, si, ar, br, sa, sb = bufs[j]
        pltpu.sync_copy(dst_hbm.at[chunk], di)
        pltpu.sync_copy(src_hbm.at[chunk], si)
        cpa = pltpu.async_copy(a_hbm.at[di], ar, sa)
        cpb = pltpu.async_copy(b_hbm.at[si], br, sb)
        return cpa, cpb

    def drain(chunk, j, cps):
        di, si, ar, br, sa, sb = bufs[j]
        cps[0].wait()
        cps[1].wait()

        def add_row(r, carry):
            for cc in range(8):
                sl = pl.ds(cc * 16, 16)
                ar[r, sl] = ar[r, sl] + br[r, sl]
            return carry

        lax.fori_loop(0, CHUNK, add_row, 0)
        pltpu.sync_copy(ar, out_hbm.at[chunk])

    def loop_body(i, carry):
        c0 = w * CPW + 2 * i
        cp0 = start(c0, 0)
        cp1 = start(c0 + 1, 1)
        drain(c0, 0, cp0)
        drain(c0 + 1, 1, cp1)
        return carry

    lax.fori_loop(0, CPW // 2, loop_body, 0)

    @pl.when(w < NCHUNK - NWORK * CPW)
    def _():
        chunk = NWORK * CPW + w
        drain(chunk, 0, start(chunk, 0))


def _sc_gather(a, b, dst_c, src_c):
    kern = pl.kernel(
        _sc_gather_body,
        out_type=jax.ShapeDtypeStruct((NCHUNK, CHUNK, D), jnp.float32),
        mesh=_sc_mesh(),
        scratch_types=[
            pltpu.VMEM((CHUNK,), jnp.int32),
            pltpu.VMEM((CHUNK,), jnp.int32),
            pltpu.VMEM((CHUNK, D), jnp.float32),
            pltpu.VMEM((CHUNK, D), jnp.float32),
            pltpu.VMEM((CHUNK,), jnp.int32),
            pltpu.VMEM((CHUNK,), jnp.int32),
            pltpu.VMEM((CHUNK, D), jnp.float32),
            pltpu.VMEM((CHUNK, D), jnp.float32),
            pltpu.SemaphoreType.DMA,
            pltpu.SemaphoreType.DMA,
            pltpu.SemaphoreType.DMA,
            pltpu.SemaphoreType.DMA,
        ],
    )
    return kern(a, b, dst_c, src_c)


def _sc_scatter_body(u_hbm, dst_hbm, out_hbm, di0, ur0, di1, ur1,
                     s0, s1, z_v, acc_sh):
    c = lax.axis_index("c")
    s = lax.axis_index("s")

    def zrow(r, carry):
        for cc in range(8):
            z_v[r, pl.ds(cc * 16, 16)] = jnp.zeros((16,), jnp.float32)
        return carry

    lax.fori_loop(0, 8, zrow, 0)
    base_rows = pl.multiple_of(s * ROWS_BIG, 8)
    nz = jnp.where(s < 15, ROWS_BIG // 8, ROWS_LAST // 8)

    def zcopy(i, carry):
        pltpu.sync_copy(z_v, acc_sh.at[pl.ds(base_rows + pl.multiple_of(i * 8, 8), 8)])
        return carry

    lax.fori_loop(0, nz, zcopy, 0)
    plsc.subcore_barrier()

    w = c * 16 + s
    bufs = ((di0, ur0, s0), (di1, ur1, s1))

    def start(chunk, j):
        di, ur, sem = bufs[j]
        pltpu.sync_copy(dst_hbm.at[chunk], di)
        return pltpu.async_copy(u_hbm.at[chunk], ur, sem)

    def drain(j, cp):
        di, ur, sem = bufs[j]
        cp.wait()
        pltpu.sync_copy(ur, acc_sh.at[di], add=True)

    def loop_body(i, carry):
        c0 = w * CPW + 2 * i
        cp0 = start(c0, 0)
        cp1 = start(c0 + 1, 1)
        drain(0, cp0)
        drain(1, cp1)
        return carry

    lax.fori_loop(0, CPW // 2, loop_body, 0)

    @pl.when(w < NCHUNK - NWORK * CPW)
    def _():
        drain(0, start(NWORK * CPW + w, 0))

    plsc.subcore_barrier()
    out_base = pl.multiple_of(c * N + s * ROWS_BIG, 8)

    @pl.when(s < 15)
    def _():
        pltpu.sync_copy(acc_sh.at[pl.ds(base_rows, ROWS_BIG)],
                        out_hbm.at[pl.ds(out_base, ROWS_BIG)])

    @pl.when(s == 15)
    def _():
        pltpu.sync_copy(acc_sh.at[pl.ds(base_rows, ROWS_LAST)],
                        out_hbm.at[pl.ds(out_base, ROWS_LAST)])


def _sc_scatter(u_c, dst_c):
    kern = pl.kernel(
        _sc_scatter_body,
        out_type=jax.ShapeDtypeStruct((2 * N, D), jnp.float32),
        mesh=_sc_mesh(),
        scratch_types=[
            pltpu.VMEM((CHUNK,), jnp.int32),
            pltpu.VMEM((CHUNK, D), jnp.float32),
            pltpu.VMEM((CHUNK,), jnp.int32),
            pltpu.VMEM((CHUNK, D), jnp.float32),
            pltpu.SemaphoreType.DMA,
            pltpu.SemaphoreType.DMA,
            pltpu.VMEM((8, D), jnp.float32),
            pltpu.VMEM_SHARED((N, D), jnp.float32),
        ],
    )
    return kern(u_c, dst_c)


def kernel(x, edge_index, edge_attr, batch, W_in, b_in, W1, b1, W2, b2,
           Wu, bu, W_pred, b_pred):
    src_c = edge_index[0].reshape(NCHUNK, CHUNK)
    dst_c = edge_index[1].reshape(NCHUNK, CHUNK)
    batch2d = batch.reshape(1, N)

    h, a, b = _node_init(x, W_in, b_in.reshape(1, D),
                         W1[0, :D, :], W1[0, D:2 * D, :])
    for l in range(L):
        gsum = _sc_gather(a, b, dst_c, src_c)
        u = _edge_mlp(gsum.reshape(E, D), edge_attr, W1[l, 2 * D:, :],
                      b1[l].reshape(1, D), W2[l], b2[l].reshape(1, D))
        parts = _sc_scatter(u.reshape(NCHUNK, CHUNK, D), dst_c)
        parts = parts.reshape(2, N, D)
        if l + 1 < L:
            h, a, b = _node_update(h, parts, Wu[l, :D, :], Wu[l, D:, :],
                                   bu[l].reshape(1, D),
                                   W1[l + 1, :D, :], W1[l + 1, D:2 * D, :])
        else:
            h = _node_update(h, parts, Wu[l, :D, :], Wu[l, D:, :],
                             bu[l].reshape(1, D), None, None)

    out = _pool(h, batch2d, W_pred.reshape(1, D), b_pred.reshape(1, 1))
    return out.reshape(-1)


# speedup vs baseline: 2.5289x; 1.0141x over previous
import functools

import jax
import jax.numpy as jnp
from jax import lax
from jax.experimental import pallas as pl
from jax.experimental.pallas import tpu as pltpu
from jax.experimental.pallas import tpu_sc as plsc

N = 10000
E = 320000
D = 128
L = 4
G = 64

CHUNK = 128
NCHUNK = E // CHUNK
NWORK = 32
CPW = NCHUNK // NWORK
CPS = (NCHUNK // 2) // 16
ROWS_BIG = 632
ROWS_LAST = N - 15 * ROWS_BIG


def _lrelu(v):
    return jnp.where(v >= 0, v, 0.01 * v)


def _node_init_body(x_ref, win_ref, bin_ref, wa_ref, wb_ref, h_ref, a_ref, b_ref):
    h = jnp.dot(x_ref[...], win_ref[...], preferred_element_type=jnp.float32)
    h = h + bin_ref[...]
    h_ref[...] = h
    a_ref[...] = jnp.dot(h, wa_ref[...], preferred_element_type=jnp.float32)
    b_ref[...] = jnp.dot(h, wb_ref[...], preferred_element_type=jnp.float32)


def _node_init(x, w_in, b_in, wa, wb):
    nb = 2000
    grid = N // nb
    blk_n = pl.BlockSpec((nb, D), lambda i: (i, 0))
    blk_w = pl.BlockSpec((D, D), lambda i: (0, 0))
    blk_b = pl.BlockSpec((1, D), lambda i: (0, 0))
    out = jax.ShapeDtypeStruct((N, D), jnp.float32)
    return pl.pallas_call(
        _node_init_body,
        grid=(grid,),
        in_specs=[blk_n, blk_w, blk_b, blk_w, blk_w],
        out_specs=[blk_n, blk_n, blk_n],
        out_shape=[out, out, out],
    )(x, w_in, b_in, wa, wb)


def _edge_body(gs_ref, ea_ref, w1c_ref, b1_ref, w2_ref, b2_ref, u_ref):
    t = gs_ref[...] + jnp.dot(ea_ref[...], w1c_ref[...],
                              preferred_element_type=jnp.float32)
    t = _lrelu(t + b1_ref[...])
    u = jnp.dot(t, w2_ref[...], preferred_element_type=jnp.float32)
    u_ref[...] = _lrelu(u + b2_ref[...])


def _edge_mlp(gsum, ea, w1c, b1l, w2l, b2l):
    be = 512
    grid = E // be
    blk_e = pl.BlockSpec((be, D), lambda i: (i, 0))
    blk_w = pl.BlockSpec((D, D), lambda i: (0, 0))
    blk_b = pl.BlockSpec((1, D), lambda i: (0, 0))
    return pl.pallas_call(
        _edge_body,
        grid=(grid,),
        in_specs=[blk_e, blk_e, blk_w, blk_b, blk_w, blk_b],
        out_specs=blk_e,
        out_shape=jax.ShapeDtypeStruct((E, D), jnp.float32),
    )(gsum, ea, w1c, b1l, w2l, b2l)


def _node_upd_body(h_ref, p_ref, wua_ref, wub_ref, bu_ref, wa_ref, wb_ref,
                   hn_ref, a_ref, b_ref):
    aggr = p_ref[0] + p_ref[1]
    upd = jnp.dot(h_ref[...], wua_ref[...], preferred_element_type=jnp.float32)
    upd = upd + jnp.dot(aggr, wub_ref[...], preferred_element_type=jnp.float32)
    hn = h_ref[...] + _lrelu(upd + bu_ref[...])
    hn_ref[...] = hn
    if a_ref is not None:
        a_ref[...] = jnp.dot(hn, wa_ref[...], preferred_element_type=jnp.float32)
        b_ref[...] = jnp.dot(hn, wb_ref[...], preferred_element_type=jnp.float32)


def _node_update(h, parts, wua, wub, bul, wa_next, wb_next):
    nb = 2000
    grid = N // nb
    blk_n = pl.BlockSpec((nb, D), lambda i: (i, 0))
    blk_p = pl.BlockSpec((2, nb, D), lambda i: (0, i, 0))
    blk_w = pl.BlockSpec((D, D), lambda i: (0, 0))
    blk_b = pl.BlockSpec((1, D), lambda i: (0, 0))
    out = jax.ShapeDtypeStruct((N, D), jnp.float32)
    if wa_next is None:
        body = lambda h_r, p_r, wua_r, wub_r, bu_r, hn_r: _node_upd_body(
            h_r, p_r, wua_r, wub_r, bu_r, None, None, hn_r, None, None)
        return pl.pallas_call(
            body,
            grid=(grid,),
            in_specs=[blk_n, blk_p, blk_w, blk_w, blk_b],
            out_specs=blk_n,
            out_shape=out,
        )(h, parts, wua, wub, bul)
    return pl.pallas_call(
        _node_upd_body,
        grid=(grid,),
        in_specs=[blk_n, blk_p, blk_w, blk_w, blk_b, blk_w, blk_w],
        out_specs=[blk_n, blk_n, blk_n],
        out_shape=[out, out, out],
    )(h, parts, wua, wub, bul, wa_next, wb_next)


def _pool_body(h_ref, batch_ref, wpt_ref, bp_ref, out_ref):
    seg = lax.broadcasted_iota(jnp.int32, (G, N), 0)
    oh = (seg == batch_ref[...]).astype(jnp.float32)
    h32 = h_ref[...]
    h_hi = h32.astype(jnp.bfloat16).astype(jnp.float32)
    h_lo = h32 - h_hi
    sums = (jnp.dot(oh, h_hi, preferred_element_type=jnp.float32)
            + jnp.dot(oh, h_lo, preferred_element_type=jnp.float32))
    cnts = jnp.sum(oh, axis=1, keepdims=True)
    hg = sums / jnp.maximum(cnts, 1.0)
    hgb = hg.astype(jnp.bfloat16).astype(jnp.float32)
    wptb = wpt_ref[...].astype(jnp.bfloat16).astype(jnp.float32)
    out_ref[...] = jnp.sum(hgb * wptb, axis=1, keepdims=True) + bp_ref[...]


def _pool(h, batch2d, wpt, bp):
    return pl.pallas_call(
        _pool_body,
        out_shape=jax.ShapeDtypeStruct((G, 1), jnp.float32),
    )(h, batch2d, wpt, bp)


def _sc_mesh():
    return plsc.VectorSubcoreMesh(core_axis_name="c", subcore_axis_name="s",
                                  num_cores=2, num_subcores=16)


def _sc_gather_body(a_hbm, b_hbm, dst_hbm, src_hbm, out_hbm,
                    di0, si0, ar0, br0, di1, si1, ar1, br1,
                    di2, si2, ar2, br2,
                    sa0, sb0, sa1, sb1, sa2, sb2):
    c = lax.axis_index("c")
    s = lax.axis_index("s")
    w = c * 16 + s

    bufs = ((di0, si0, ar0, br0, sa0, sb0), (di1, si1, ar1, br1, sa1, sb1),
            (di2, si2, ar2, br2, sa2, sb2))

    def start(chunk, j):
        di, si, ar, br, sa, sb = bufs[j]
        pltpu.sync_copy(dst_hbm.at[chunk], di)
        pltpu.sync_copy(src_hbm.at[chunk], si)
        cpa = pltpu.async_copy(a_hbm.at[di], ar, sa)
        cpb = pltpu.async_copy(b_hbm.at[si], br, sb)
        return cpa, cpb

    def drain(chunk, j, cps):
        di, si, ar, br, sa, sb = bufs[j]
        cps[0].wait()
        cps[1].wait()

        def add_row(r, carry):
            for cc in range(8):
                sl = pl.ds(cc * 16, 16)
                ar[r, sl] = ar[r, sl] + br[r, sl]
            return carry

        lax.fori_loop(0, CHUNK, add_row, 0)
        pltpu.sync_copy(ar, out_hbm.at[chunk])

    def loop_body(i, carry):
        c0 = w * CPW + 3 * i
        cp0 = start(c0, 0)
        cp1 = start(c0 + 1, 1)
        cp2 = start(c0 + 2, 2)
        drain(c0, 0, cp0)
        drain(c0 + 1, 1, cp1)
        drain(c0 + 2, 2, cp2)
        return carry

    lax.fori_loop(0, CPW // 3, loop_body, 0)

    @pl.when(w < NCHUNK - NWORK * CPW)
    def _():
        chunk = NWORK * CPW + w
        drain(chunk, 0, start(chunk, 0))


def _sc_gather(a, b, dst_c, src_c):
    kern = pl.kernel(
        _sc_gather_body,
        out_type=jax.ShapeDtypeStruct((NCHUNK, CHUNK, D), jnp.float32),
        mesh=_sc_mesh(),
        scratch_types=[
            pltpu.VMEM((CHUNK,), jnp.int32),
            pltpu.VMEM((CHUNK,), jnp.int32),
            pltpu.VMEM((CHUNK, D), jnp.float32),
            pltpu.VMEM((CHUNK, D), jnp.float32),
            pltpu.VMEM((CHUNK,), jnp.int32),
            pltpu.VMEM((CHUNK,), jnp.int32),
            pltpu.VMEM((CHUNK, D), jnp.float32),
            pltpu.VMEM((CHUNK, D), jnp.float32),
            pltpu.VMEM((CHUNK,), jnp.int32),
            pltpu.VMEM((CHUNK,), jnp.int32),
            pltpu.VMEM((CHUNK, D), jnp.float32),
            pltpu.VMEM((CHUNK, D), jnp.float32),
            pltpu.SemaphoreType.DMA,
            pltpu.SemaphoreType.DMA,
            pltpu.SemaphoreType.DMA,
            pltpu.SemaphoreType.DMA,
            pltpu.SemaphoreType.DMA,
            pltpu.SemaphoreType.DMA,
        ],
    )
    return kern(a, b, dst_c, src_c)


def _sc_scatter_body(u_hbm, dst_hbm, out_hbm, di0, ur0, di1, ur1,
                     s0, s1, z_v, acc_sh):
    c = lax.axis_index("c")
    s = lax.axis_index("s")

    def zrow(r, carry):
        for cc in range(8):
            z_v[r, pl.ds(cc * 16, 16)] = jnp.zeros((16,), jnp.float32)
        return carry

    lax.fori_loop(0, 40, zrow, 0)
    base_rows = pl.multiple_of(s * ROWS_BIG, 8)
    nz = jnp.where(s < 15, ROWS_BIG // 40, ROWS_LAST // 40)

    def zcopy(i, carry):
        pltpu.sync_copy(z_v, acc_sh.at[pl.ds(base_rows + pl.multiple_of(i * 40, 8), 40)])
        return carry

    lax.fori_loop(0, nz, zcopy, 0)

    @pl.when(s < 15)
    def _():
        pltpu.sync_copy(z_v.at[pl.ds(0, 32)],
                        acc_sh.at[pl.ds(base_rows + 600, 32)])

    plsc.subcore_barrier()

    w = c * 16 + s
    bufs = ((di0, ur0, s0), (di1, ur1, s1))

    def start(chunk, j):
        di, ur, sem = bufs[j]
        pltpu.sync_copy(dst_hbm.at[chunk], di)
        return pltpu.async_copy(u_hbm.at[chunk], ur, sem)

    def drain(j, cp):
        di, ur, sem = bufs[j]
        cp.wait()
        pltpu.sync_copy(ur, acc_sh.at[di], add=True)

    def loop_body(i, carry):
        c0 = w * CPW + 2 * i
        cp0 = start(c0, 0)
        cp1 = start(c0 + 1, 1)
        drain(0, cp0)
        drain(1, cp1)
        return carry

    lax.fori_loop(0, CPW // 2, loop_body, 0)

    @pl.when(w < NCHUNK - NWORK * CPW)
    def _():
        drain(0, start(NWORK * CPW + w, 0))

    plsc.subcore_barrier()
    out_base = pl.multiple_of(c * N + s * ROWS_BIG, 8)

    @pl.when(s < 15)
    def _():
        pltpu.sync_copy(acc_sh.at[pl.ds(base_rows, ROWS_BIG)],
                        out_hbm.at[pl.ds(out_base, ROWS_BIG)])

    @pl.when(s == 15)
    def _():
        pltpu.sync_copy(acc_sh.at[pl.ds(base_rows, ROWS_LAST)],
                        out_hbm.at[pl.ds(out_base, ROWS_LAST)])


def _sc_scatter(u_c, dst_c):
    kern = pl.kernel(
        _sc_scatter_body,
        out_type=jax.ShapeDtypeStruct((2 * N, D), jnp.float32),
        mesh=_sc_mesh(),
        scratch_types=[
            pltpu.VMEM((CHUNK,), jnp.int32),
            pltpu.VMEM((CHUNK, D), jnp.float32),
            pltpu.VMEM((CHUNK,), jnp.int32),
            pltpu.VMEM((CHUNK, D), jnp.float32),
            pltpu.SemaphoreType.DMA,
            pltpu.SemaphoreType.DMA,
            pltpu.VMEM((40, D), jnp.float32),
            pltpu.VMEM_SHARED((N, D), jnp.float32),
        ],
    )
    return kern(u_c, dst_c)


def kernel(x, edge_index, edge_attr, batch, W_in, b_in, W1, b1, W2, b2,
           Wu, bu, W_pred, b_pred):
    src_c = edge_index[0].reshape(NCHUNK, CHUNK)
    dst_c = edge_index[1].reshape(NCHUNK, CHUNK)
    batch2d = batch.reshape(1, N)

    h, a, b = _node_init(x, W_in, b_in.reshape(1, D),
                         W1[0, :D, :], W1[0, D:2 * D, :])
    for l in range(L):
        gsum = _sc_gather(a, b, dst_c, src_c)
        u = _edge_mlp(gsum.reshape(E, D), edge_attr, W1[l, 2 * D:, :],
                      b1[l].reshape(1, D), W2[l], b2[l].reshape(1, D))
        parts = _sc_scatter(u.reshape(NCHUNK, CHUNK, D), dst_c)
        parts = parts.reshape(2, N, D)
        if l + 1 < L:
            h, a, b = _node_update(h, parts, Wu[l, :D, :], Wu[l, D:, :],
                                   bu[l].reshape(1, D),
                                   W1[l + 1, :D, :], W1[l + 1, D:2 * D, :])
        else:
            h = _node_update(h, parts, Wu[l, :D, :], Wu[l, D:, :],
                             bu[l].reshape(1, D), None, None)

    out = _pool(h, batch2d, W_pred.reshape(1, D), b_pred.reshape(1, 1))
    return out.reshape(-1)


# speedup vs baseline: 3.3133x; 1.3102x over previous
import functools

import jax
import jax.numpy as jnp
from jax import lax
from jax.experimental import pallas as pl
from jax.experimental.pallas import tpu as pltpu
from jax.experimental.pallas import tpu_sc as plsc

N = 10000
E = 320000
D = 128
L = 4
G = 64

CHUNK = 128
NCHUNK = E // CHUNK
NWORK = 32
HCHUNK = NCHUNK // 2
HCPW = HCHUNK // NWORK
HREM = HCHUNK - NWORK * HCPW
ROWS_BIG = 632
ROWS_LAST = N - 15 * ROWS_BIG


def _lrelu(v):
    return jnp.where(v >= 0, v, 0.01 * v)


def _node_init_body(x_ref, win_ref, bin_ref, wa_ref, wb_ref, h_ref, a_ref, b_ref):
    h = jnp.dot(x_ref[...], win_ref[...], preferred_element_type=jnp.float32)
    h = h + bin_ref[...]
    h_ref[...] = h
    a_ref[...] = jnp.dot(h, wa_ref[...], preferred_element_type=jnp.float32)
    b_ref[...] = jnp.dot(h, wb_ref[...], preferred_element_type=jnp.float32)


def _node_init(x, w_in, b_in, wa, wb):
    nb = 2000
    grid = N // nb
    blk_n = pl.BlockSpec((nb, D), lambda i: (i, 0))
    blk_w = pl.BlockSpec((D, D), lambda i: (0, 0))
    blk_b = pl.BlockSpec((1, D), lambda i: (0, 0))
    out = jax.ShapeDtypeStruct((N, D), jnp.float32)
    return pl.pallas_call(
        _node_init_body,
        grid=(grid,),
        in_specs=[blk_n, blk_w, blk_b, blk_w, blk_w],
        out_specs=[blk_n, blk_n, blk_n],
        out_shape=[out, out, out],
    )(x, w_in, b_in, wa, wb)


def _edge_body(gs_ref, ea_ref, w1c_ref, b1_ref, w2_ref, b2_ref, u_ref):
    t = gs_ref[...] + jnp.dot(ea_ref[...], w1c_ref[...],
                              preferred_element_type=jnp.float32)
    t = _lrelu(t + b1_ref[...])
    u = jnp.dot(t, w2_ref[...], preferred_element_type=jnp.float32)
    u_ref[...] = _lrelu(u + b2_ref[...])


def _edge_mlp(gsum_h, ea, w1c, b1l, w2l, b2l, half):
    be = 640
    grid = (E // 2) // be
    off = half * grid
    blk_h = pl.BlockSpec((be, D), lambda i: (i, 0))
    blk_e = pl.BlockSpec((be, D), lambda i: (i + off, 0))
    blk_w = pl.BlockSpec((D, D), lambda i: (0, 0))
    blk_b = pl.BlockSpec((1, D), lambda i: (0, 0))
    return pl.pallas_call(
        _edge_body,
        grid=(grid,),
        in_specs=[blk_h, blk_e, blk_w, blk_b, blk_w, blk_b],
        out_specs=blk_h,
        out_shape=jax.ShapeDtypeStruct((E // 2, D), jnp.float32),
    )(gsum_h, ea, w1c, b1l, w2l, b2l)


def _node_upd_body(h_ref, p_ref, q_ref, wua_ref, wub_ref, bu_ref, wa_ref, wb_ref,
                   hn_ref, a_ref, b_ref):
    aggr = p_ref[0] + p_ref[1] + q_ref[0] + q_ref[1]
    upd = jnp.dot(h_ref[...], wua_ref[...], preferred_element_type=jnp.float32)
    upd = upd + jnp.dot(aggr, wub_ref[...], preferred_element_type=jnp.float32)
    hn = h_ref[...] + _lrelu(upd + bu_ref[...])
    hn_ref[...] = hn
    if a_ref is not None:
        a_ref[...] = jnp.dot(hn, wa_ref[...], preferred_element_type=jnp.float32)
        b_ref[...] = jnp.dot(hn, wb_ref[...], preferred_element_type=jnp.float32)


def _node_update(h, parts0, parts1, wua, wub, bul, wa_next, wb_next):
    nb = 2000
    grid = N // nb
    blk_n = pl.BlockSpec((nb, D), lambda i: (i, 0))
    blk_p = pl.BlockSpec((2, nb, D), lambda i: (0, i, 0))
    blk_w = pl.BlockSpec((D, D), lambda i: (0, 0))
    blk_b = pl.BlockSpec((1, D), lambda i: (0, 0))
    out = jax.ShapeDtypeStruct((N, D), jnp.float32)
    if wa_next is None:
        body = lambda h_r, p_r, q_r, wua_r, wub_r, bu_r, hn_r: _node_upd_body(
            h_r, p_r, q_r, wua_r, wub_r, bu_r, None, None, hn_r, None, None)
        return pl.pallas_call(
            body,
            grid=(grid,),
            in_specs=[blk_n, blk_p, blk_p, blk_w, blk_w, blk_b],
            out_specs=blk_n,
            out_shape=out,
        )(h, parts0, parts1, wua, wub, bul)
    return pl.pallas_call(
        _node_upd_body,
        grid=(grid,),
        in_specs=[blk_n, blk_p, blk_p, blk_w, blk_w, blk_b, blk_w, blk_w],
        out_specs=[blk_n, blk_n, blk_n],
        out_shape=[out, out, out],
    )(h, parts0, parts1, wua, wub, bul, wa_next, wb_next)


def _pool_body(h_ref, batch_ref, wpt_ref, bp_ref, out_ref):
    seg = lax.broadcasted_iota(jnp.int32, (G, N), 0)
    oh = (seg == batch_ref[...]).astype(jnp.float32)
    h32 = h_ref[...]
    h_hi = h32.astype(jnp.bfloat16).astype(jnp.float32)
    h_lo = h32 - h_hi
    sums = (jnp.dot(oh, h_hi, preferred_element_type=jnp.float32)
            + jnp.dot(oh, h_lo, preferred_element_type=jnp.float32))
    cnts = jnp.sum(oh, axis=1, keepdims=True)
    hg = sums / jnp.maximum(cnts, 1.0)
    hgb = hg.astype(jnp.bfloat16).astype(jnp.float32)
    wptb = wpt_ref[...].astype(jnp.bfloat16).astype(jnp.float32)
    out_ref[...] = jnp.sum(hgb * wptb, axis=1, keepdims=True) + bp_ref[...]


def _pool(h, batch2d, wpt, bp):
    return pl.pallas_call(
        _pool_body,
        out_shape=jax.ShapeDtypeStruct((G, 1), jnp.float32),
    )(h, batch2d, wpt, bp)


def _sc_mesh():
    return plsc.VectorSubcoreMesh(core_axis_name="c", subcore_axis_name="s",
                                  num_cores=2, num_subcores=16)


def _sc_gather_body(half, a_hbm, b_hbm, dst_hbm, src_hbm, out_hbm,
                    di0, si0, ar0, br0, di1, si1, ar1, br1,
                    di2, si2, ar2, br2,
                    sa0, sb0, sa1, sb1, sa2, sb2):
    c = lax.axis_index("c")
    s = lax.axis_index("s")
    w = c * 16 + s
    hbase = half * HCHUNK

    bufs = ((di0, si0, ar0, br0, sa0, sb0), (di1, si1, ar1, br1, sa1, sb1),
            (di2, si2, ar2, br2, sa2, sb2))

    def start(chunk, j):
        di, si, ar, br, sa, sb = bufs[j]
        pltpu.sync_copy(dst_hbm.at[chunk], di)
        pltpu.sync_copy(src_hbm.at[chunk], si)
        cpa = pltpu.async_copy(a_hbm.at[di], ar, sa)
        cpb = pltpu.async_copy(b_hbm.at[si], br, sb)
        return cpa, cpb

    def drain(chunk, j, cps):
        di, si, ar, br, sa, sb = bufs[j]
        cps[0].wait()
        cps[1].wait()

        def add_row(r, carry):
            for cc in range(8):
                sl = pl.ds(cc * 16, 16)
                ar[r, sl] = ar[r, sl] + br[r, sl]
            return carry

        lax.fori_loop(0, CHUNK, add_row, 0)
        pltpu.sync_copy(ar, out_hbm.at[chunk - hbase])

    def loop_body(i, carry):
        c0 = hbase + w * HCPW + 3 * i
        cp0 = start(c0, 0)
        cp1 = start(c0 + 1, 1)
        cp2 = start(c0 + 2, 2)
        drain(c0, 0, cp0)
        drain(c0 + 1, 1, cp1)
        drain(c0 + 2, 2, cp2)
        return carry

    lax.fori_loop(0, HCPW // 3, loop_body, 0)

    @pl.when(w < HREM)
    def _():
        chunk = hbase + NWORK * HCPW + w
        drain(chunk, 0, start(chunk, 0))


def _sc_gather(a, b, dst_c, src_c, half):
    kern = pl.kernel(
        functools.partial(_sc_gather_body, half),
        out_type=jax.ShapeDtypeStruct((HCHUNK, CHUNK, D), jnp.float32),
        mesh=_sc_mesh(),
        scratch_types=[
            pltpu.VMEM((CHUNK,), jnp.int32),
            pltpu.VMEM((CHUNK,), jnp.int32),
            pltpu.VMEM((CHUNK, D), jnp.float32),
            pltpu.VMEM((CHUNK, D), jnp.float32),
            pltpu.VMEM((CHUNK,), jnp.int32),
            pltpu.VMEM((CHUNK,), jnp.int32),
            pltpu.VMEM((CHUNK, D), jnp.float32),
            pltpu.VMEM((CHUNK, D), jnp.float32),
            pltpu.VMEM((CHUNK,), jnp.int32),
            pltpu.VMEM((CHUNK,), jnp.int32),
            pltpu.VMEM((CHUNK, D), jnp.float32),
            pltpu.VMEM((CHUNK, D), jnp.float32),
            pltpu.SemaphoreType.DMA,
            pltpu.SemaphoreType.DMA,
            pltpu.SemaphoreType.DMA,
            pltpu.SemaphoreType.DMA,
            pltpu.SemaphoreType.DMA,
            pltpu.SemaphoreType.DMA,
        ],
    )
    return kern(a, b, dst_c, src_c)


def _sc_scatter_body(half, u_hbm, dst_hbm, out_hbm, di0, ur0, di1, ur1,
                     s0, s1, z_v, acc_sh):
    c = lax.axis_index("c")
    s = lax.axis_index("s")
    hbase = half * HCHUNK

    def zrow(r, carry):
        for cc in range(8):
            z_v[r, pl.ds(cc * 16, 16)] = jnp.zeros((16,), jnp.float32)
        return carry

    lax.fori_loop(0, 40, zrow, 0)
    base_rows = pl.multiple_of(s * ROWS_BIG, 8)
    nz = jnp.where(s < 15, ROWS_BIG // 40, ROWS_LAST // 40)

    def zcopy(i, carry):
        pltpu.sync_copy(z_v, acc_sh.at[pl.ds(base_rows + pl.multiple_of(i * 40, 8), 40)])
        return carry

    lax.fori_loop(0, nz, zcopy, 0)

    @pl.when(s < 15)
    def _():
        pltpu.sync_copy(z_v.at[pl.ds(0, 32)],
                        acc_sh.at[pl.ds(base_rows + 600, 32)])

    plsc.subcore_barrier()

    w = c * 16 + s
    bufs = ((di0, ur0, s0), (di1, ur1, s1))

    def start(chunk, j):
        di, ur, sem = bufs[j]
        pltpu.sync_copy(dst_hbm.at[chunk], di)
        return pltpu.async_copy(u_hbm.at[chunk - hbase], ur, sem)

    def drain(j, cp):
        di, ur, sem = bufs[j]
        cp.wait()
        pltpu.sync_copy(ur, acc_sh.at[di], add=True)

    def loop_body(i, carry):
        c0 = hbase + w * HCPW + 2 * i
        cp0 = start(c0, 0)
        cp1 = start(c0 + 1, 1)
        drain(0, cp0)
        drain(1, cp1)
        return carry

    lax.fori_loop(0, HCPW // 2, loop_body, 0)
    drain(0, start(hbase + w * HCPW + HCPW - 1, 0))

    @pl.when(w < HREM)
    def _():
        drain(0, start(hbase + NWORK * HCPW + w, 0))

    plsc.subcore_barrier()
    out_base = pl.multiple_of(c * N + s * ROWS_BIG, 8)

    @pl.when(s < 15)
    def _():
        pltpu.sync_copy(acc_sh.at[pl.ds(base_rows, ROWS_BIG)],
                        out_hbm.at[pl.ds(out_base, ROWS_BIG)])

    @pl.when(s == 15)
    def _():
        pltpu.sync_copy(acc_sh.at[pl.ds(base_rows, ROWS_LAST)],
                        out_hbm.at[pl.ds(out_base, ROWS_LAST)])


def _sc_scatter(u_c, dst_c, half):
    kern = pl.kernel(
        functools.partial(_sc_scatter_body, half),
        out_type=jax.ShapeDtypeStruct((2 * N, D), jnp.float32),
        mesh=_sc_mesh(),
        scratch_types=[
            pltpu.VMEM((CHUNK,), jnp.int32),
            pltpu.VMEM((CHUNK, D), jnp.float32),
            pltpu.VMEM((CHUNK,), jnp.int32),
            pltpu.VMEM((CHUNK, D), jnp.float32),
            pltpu.SemaphoreType.DMA,
            pltpu.SemaphoreType.DMA,
            pltpu.VMEM((40, D), jnp.float32),
            pltpu.VMEM_SHARED((N, D), jnp.float32),
        ],
    )
    return kern(u_c, dst_c)


def kernel(x, edge_index, edge_attr, batch, W_in, b_in, W1, b1, W2, b2,
           Wu, bu, W_pred, b_pred):
    src_c = edge_index[0].reshape(NCHUNK, CHUNK)
    dst_c = edge_index[1].reshape(NCHUNK, CHUNK)
    batch2d = batch.reshape(1, N)

    h, a, b = _node_init(x, W_in, b_in.reshape(1, D),
                         W1[0, :D, :], W1[0, D:2 * D, :])
    for l in range(L):
        w1c = W1[l, 2 * D:, :]
        b1l = b1[l].reshape(1, D)
        b2l = b2[l].reshape(1, D)
        gs0 = _sc_gather(a, b, dst_c, src_c, 0)
        u0 = _edge_mlp(gs0.reshape(E // 2, D), edge_attr, w1c, b1l, W2[l], b2l, 0)
        gs1 = _sc_gather(a, b, dst_c, src_c, 1)
        u1 = _edge_mlp(gs1.reshape(E // 2, D), edge_attr, w1c, b1l, W2[l], b2l, 1)
        p0 = _sc_scatter(u0.reshape(HCHUNK, CHUNK, D), dst_c, 0).reshape(2, N, D)
        p1 = _sc_scatter(u1.reshape(HCHUNK, CHUNK, D), dst_c, 1).reshape(2, N, D)
        if l + 1 < L:
            h, a, b = _node_update(h, p0, p1, Wu[l, :D, :], Wu[l, D:, :],
                                   bu[l].reshape(1, D),
                                   W1[l + 1, :D, :], W1[l + 1, D:2 * D, :])
        else:
            h = _node_update(h, p0, p1, Wu[l, :D, :], Wu[l, D:, :],
                             bu[l].reshape(1, D), None, None)

    out = _pool(h, batch2d, W_pred.reshape(1, D), b_pred.reshape(1, 1))
    return out.reshape(-1)
